# Initial kernel scaffold; baseline (speedup 1.0000x reference)
import jax
import jax.numpy as jnp
from jax.experimental import pallas as pl


def kernel(user_emb, item_emb, edge_index, edge_values):
    raise NotImplementedError("write your pallas kernel here")


# speedup vs baseline: 2.6600x; 2.6600x over previous
import functools

import jax
import jax.numpy as jnp
from jax import lax
from jax.experimental import pallas as pl
from jax.experimental.pallas import tpu as pltpu
from jax.experimental.pallas import tpu_sc as plsc

N_USERS = 25000
N_ITEMS = 25000
N_NODES = N_USERS + N_ITEMS
D = 64
E = 800000
HALF = N_NODES // 2
N_SUB = 16
ROWS_PER_SUB = 1600
ACC_ROWS = N_SUB * ROWS_PER_SUB
DUMMY_ROW = 25500
K = 128
CHUNKS_PER_SUB = 392
EDGES_PER_SUB = CHUNKS_PER_SUB * K
E_PAD = N_SUB * EDGES_PER_SUB

RESIDUAL_COFF = 0.1

P0 = 7.8579951959925864
P1 = -22.224249412937251
P2 = 29.508311855171733
P3 = -24.651553647411124
P4 = 14.776720780182517
P5 = -6.8931074954875129


def _hsum_bcast(p):
    lanes = lax.iota(jnp.int32, 16)
    for sh in (8, 4, 2, 1):
        idx = (lanes + sh) & 15
        p = p + p.at[idx].get(mode="promise_in_bounds")
    return p


def _l2norm(x, eps=1e-12):
    n = jnp.linalg.norm(x, axis=-1, keepdims=True)
    return x / jnp.maximum(n, eps)


def _edge_body(emb, srci, dsti, ev, zrows, out,
               sidx_v, didx_v, ev_v, scat_v, srows, drows, acc, sem_a, sem_b):
    c = lax.axis_index("c")
    s = lax.axis_index("s")

    pltpu.sync_copy(zrows, acc.at[pl.ds(s * ROWS_PER_SUB, ROWS_PER_SUB)])
    plsc.subcore_barrier()

    def chunk_body(j, _):
        base = (s * CHUNKS_PER_SUB + j) * K
        pltpu.sync_copy(srci.at[pl.ds(base, K)], sidx_v)
        pltpu.sync_copy(dsti.at[pl.ds(base, K)], didx_v)
        pltpu.sync_copy(ev.at[pl.ds(base, K)], ev_v)
        ga = pltpu.async_copy(emb.at[sidx_v], srows, sem_a)
        gb = pltpu.async_copy(emb.at[didx_v], drows, sem_b)
        ga.wait()
        gb.wait()

        def group_body(g, _):
            evg = ev_v[pl.ds(g * 16, 16)]
            for lane in range(16):
                e = g * 16 + lane
                a0 = srows[e, pl.ds(0, 16)]
                a1 = srows[e, pl.ds(16, 16)]
                a2 = srows[e, pl.ds(32, 16)]
                a3 = srows[e, pl.ds(48, 16)]
                b0 = drows[e, pl.ds(0, 16)]
                b1 = drows[e, pl.ds(16, 16)]
                b2 = drows[e, pl.ds(32, 16)]
                b3 = drows[e, pl.ds(48, 16)]
                p = a0 * b0 + a1 * b1 + a2 * b2 + a3 * b3
                dot = _hsum_bcast(p)
                cm = dot * (1.0 / 64.0)
                w = ((((P5 * cm + P4) * cm + P3) * cm + P2) * cm + P1) * cm + P0
                w = w * evg[lane]
                drows[e, pl.ds(0, 16)] = b0 * w
                drows[e, pl.ds(16, 16)] = b1 * w
                drows[e, pl.ds(32, 16)] = b2 * w
                drows[e, pl.ds(48, 16)] = b3 * w
            return 0

        lax.fori_loop(0, K // 16, group_body, 0)

        def clamp_body(g, _):
            v = sidx_v[pl.ds(g * 16, 16)]
            local = v - c * HALF
            ok = (local >= 0) & (local < HALF)
            scat_v[pl.ds(g * 16, 16)] = jnp.where(ok, local, DUMMY_ROW)
            return 0

        lax.fori_loop(0, K // 16, clamp_body, 0)
        pltpu.sync_copy(drows, acc.at[scat_v], add=True)
        return 0

    lax.fori_loop(0, CHUNKS_PER_SUB, chunk_body, 0)
    plsc.subcore_barrier()
    pltpu.sync_copy(acc.at[pl.ds(s * ROWS_PER_SUB, ROWS_PER_SUB)],
                    out.at[c, pl.ds(s * ROWS_PER_SUB, ROWS_PER_SUB)])


@jax.jit
def _edge_phase(emb, srci, dsti, ev, zrows):
    mesh = plsc.VectorSubcoreMesh(core_axis_name="c", subcore_axis_name="s")
    fn = pl.kernel(
        _edge_body,
        mesh=mesh,
        compiler_params=pltpu.CompilerParams(use_tc_tiling_on_sc=False),
        out_type=jax.ShapeDtypeStruct((2, ACC_ROWS, D), jnp.float32),
        scratch_types=[
            pltpu.VMEM((K,), jnp.int32),
            pltpu.VMEM((K,), jnp.int32),
            pltpu.VMEM((K,), jnp.float32),
            pltpu.VMEM((K,), jnp.int32),
            pltpu.VMEM((K, D), jnp.float32),
            pltpu.VMEM((K, D), jnp.float32),
            pltpu.VMEM_SHARED((ACC_ROWS, D), jnp.float32),
            pltpu.SemaphoreType.DMA,
            pltpu.SemaphoreType.DMA,
        ],
    )
    return fn(emb, srci, dsti, ev, zrows)


def kernel(user_emb, item_emb, edge_index, edge_values):
    all_emb = jnp.concatenate([user_emb, item_emb], axis=0)
    initial_emb = _l2norm(all_emb)

    pad = E_PAD - E
    srcp = jnp.concatenate([edge_index[0], jnp.zeros((pad,), jnp.int32)])
    dstp = jnp.concatenate([edge_index[1], jnp.zeros((pad,), jnp.int32)])
    evp = jnp.concatenate([edge_values, jnp.zeros((pad,), jnp.float32)])
    zrows = jnp.zeros((ROWS_PER_SUB, D), jnp.float32)

    emb = all_emb
    emb_sum = all_emb
    for _ in range(3):
        emb = _l2norm(emb + RESIDUAL_COFF * initial_emb)
        acc = _edge_phase(emb, srcp, dstp, evp, zrows)
        neighbor = jnp.concatenate([acc[0, :HALF], acc[1, :HALF]], axis=0)
        emb = neighbor + RESIDUAL_COFF * (emb - initial_emb)
        emb_sum = emb_sum + emb
    light_out = emb_sum * 0.25
    return (light_out[:N_USERS], light_out[N_USERS:])


# speedup vs baseline: 3.1697x; 1.1916x over previous
import functools

import jax
import jax.numpy as jnp
from jax import lax
from jax.experimental import pallas as pl
from jax.experimental.pallas import tpu as pltpu
from jax.experimental.pallas import tpu_sc as plsc

N_USERS = 25000
N_ITEMS = 25000
N_NODES = N_USERS + N_ITEMS
D = 64
E = 800000
HALF = N_NODES // 2
N_SUB = 16
ROWS_PER_SUB = 1563
ACC_ROWS = N_SUB * ROWS_PER_SUB
DUMMY_ROW = 25004
K = 96
CHUNKS_PER_SUB = 522
EDGES_PER_SUB = CHUNKS_PER_SUB * K
E_PAD = N_SUB * EDGES_PER_SUB

RESIDUAL_COFF = 0.1

P0 = 7.8579951959925864
P1 = -22.224249412937251
P2 = 29.508311855171733
P3 = -24.651553647411124
P4 = 14.776720780182517
P5 = -6.8931074954875129


def _hsum_bcast(p):
    lanes = lax.iota(jnp.int32, 16)
    for sh in (8, 4, 2, 1):
        idx = (lanes + sh) & 15
        p = p + p.at[idx].get(mode="promise_in_bounds")
    return p


def _l2norm(x, eps=1e-12):
    n = jnp.linalg.norm(x, axis=-1, keepdims=True)
    return x / jnp.maximum(n, eps)


def _edge_body(emb, srci, dsti, ev, zrows, out,
               sidx0, didx0, ev0, srows0, drows0,
               sidx1, didx1, ev1, srows1, drows1,
               scat_v, acc, sa0, sb0, sa1, sb1):
    c = lax.axis_index("c")
    s = lax.axis_index("s")
    bufs = ((sidx0, didx0, ev0, srows0, drows0, sa0, sb0),
            (sidx1, didx1, ev1, srows1, drows1, sa1, sb1))

    pltpu.sync_copy(zrows, acc.at[pl.ds(s * ROWS_PER_SUB, ROWS_PER_SUB)])
    plsc.subcore_barrier()

    def issue(j, buf):
        sidx_v, didx_v, ev_v, srows, drows, sem_a, sem_b = buf
        base = (s * CHUNKS_PER_SUB + j) * K
        pltpu.sync_copy(srci.at[pl.ds(base, K)], sidx_v)
        pltpu.sync_copy(dsti.at[pl.ds(base, K)], didx_v)
        pltpu.sync_copy(ev.at[pl.ds(base, K)], ev_v)
        pltpu.async_copy(emb.at[sidx_v], srows, sem_a)
        pltpu.async_copy(emb.at[didx_v], drows, sem_b)

    def drain(buf):
        sidx_v, didx_v, ev_v, srows, drows, sem_a, sem_b = buf
        pltpu.make_async_copy(emb.at[sidx_v], srows, sem_a).wait()
        pltpu.make_async_copy(emb.at[didx_v], drows, sem_b).wait()

    def compute_scatter(buf):
        sidx_v, didx_v, ev_v, srows, drows, sem_a, sem_b = buf

        def group_body(g, _):
            evg = ev_v[pl.ds(g * 16, 16)]
            for lane in range(16):
                e = g * 16 + lane
                a0 = srows[e, pl.ds(0, 16)]
                a1 = srows[e, pl.ds(16, 16)]
                a2 = srows[e, pl.ds(32, 16)]
                a3 = srows[e, pl.ds(48, 16)]
                b0 = drows[e, pl.ds(0, 16)]
                b1 = drows[e, pl.ds(16, 16)]
                b2 = drows[e, pl.ds(32, 16)]
                b3 = drows[e, pl.ds(48, 16)]
                p = a0 * b0 + a1 * b1 + a2 * b2 + a3 * b3
                dot = _hsum_bcast(p)
                cm = dot * (1.0 / 64.0)
                w = ((((P5 * cm + P4) * cm + P3) * cm + P2) * cm + P1) * cm + P0
                w = w * evg[lane]
                drows[e, pl.ds(0, 16)] = b0 * w
                drows[e, pl.ds(16, 16)] = b1 * w
                drows[e, pl.ds(32, 16)] = b2 * w
                drows[e, pl.ds(48, 16)] = b3 * w
            return 0

        lax.fori_loop(0, K // 16, group_body, 0)

        def clamp_body(g, _):
            v = sidx_v[pl.ds(g * 16, 16)]
            local = v - c * HALF
            ok = (local >= 0) & (local < HALF)
            scat_v[pl.ds(g * 16, 16)] = jnp.where(ok, local, DUMMY_ROW)
            return 0

        lax.fori_loop(0, K // 16, clamp_body, 0)
        pltpu.sync_copy(drows, acc.at[scat_v], add=True)

    last = CHUNKS_PER_SUB - 1
    issue(0, bufs[0])

    def pair_body(i, _):
        j = i * 2
        drain(bufs[0])
        issue(j + 1, bufs[1])
        compute_scatter(bufs[0])
        drain(bufs[1])
        issue(jnp.minimum(j + 2, last), bufs[0])
        compute_scatter(bufs[1])
        return 0

    lax.fori_loop(0, CHUNKS_PER_SUB // 2, pair_body, 0)
    drain(bufs[0])
    plsc.subcore_barrier()
    pltpu.sync_copy(acc.at[pl.ds(s * ROWS_PER_SUB, ROWS_PER_SUB)],
                    out.at[c, pl.ds(s * ROWS_PER_SUB, ROWS_PER_SUB)])


@jax.jit
def _edge_phase(emb, srci, dsti, ev, zrows):
    mesh = plsc.VectorSubcoreMesh(core_axis_name="c", subcore_axis_name="s")
    fn = pl.kernel(
        _edge_body,
        mesh=mesh,
        compiler_params=pltpu.CompilerParams(use_tc_tiling_on_sc=False),
        out_type=jax.ShapeDtypeStruct((2, ACC_ROWS, D), jnp.float32),
        scratch_types=[
            pltpu.VMEM((K,), jnp.int32),
            pltpu.VMEM((K,), jnp.int32),
            pltpu.VMEM((K,), jnp.float32),
            pltpu.VMEM((K, D), jnp.float32),
            pltpu.VMEM((K, D), jnp.float32),
            pltpu.VMEM((K,), jnp.int32),
            pltpu.VMEM((K,), jnp.int32),
            pltpu.VMEM((K,), jnp.float32),
            pltpu.VMEM((K, D), jnp.float32),
            pltpu.VMEM((K, D), jnp.float32),
            pltpu.VMEM((K,), jnp.int32),
            pltpu.VMEM_SHARED((ACC_ROWS, D), jnp.float32),
            pltpu.SemaphoreType.DMA,
            pltpu.SemaphoreType.DMA,
            pltpu.SemaphoreType.DMA,
            pltpu.SemaphoreType.DMA,
        ],
    )
    return fn(emb, srci, dsti, ev, zrows)


def kernel(user_emb, item_emb, edge_index, edge_values):
    all_emb = jnp.concatenate([user_emb, item_emb], axis=0)
    initial_emb = _l2norm(all_emb)

    pad = E_PAD - E
    srcp = jnp.concatenate([edge_index[0], jnp.zeros((pad,), jnp.int32)])
    dstp = jnp.concatenate([edge_index[1], jnp.zeros((pad,), jnp.int32)])
    evp = jnp.concatenate([edge_values, jnp.zeros((pad,), jnp.float32)])
    zrows = jnp.zeros((ROWS_PER_SUB, D), jnp.float32)

    emb = all_emb
    emb_sum = all_emb
    for _ in range(3):
        emb = _l2norm(emb + RESIDUAL_COFF * initial_emb)
        acc = _edge_phase(emb, srcp, dstp, evp, zrows)
        neighbor = jnp.concatenate([acc[0, :HALF], acc[1, :HALF]], axis=0)
        emb = neighbor + RESIDUAL_COFF * (emb - initial_emb)
        emb_sum = emb_sum + emb
    light_out = emb_sum * 0.25
    return (light_out[:N_USERS], light_out[N_USERS:])
